# baseline (speedup 1.0000x reference)
import functools

import jax
import jax.numpy as jnp
from jax import lax
from jax.experimental import pallas as pl
from jax.experimental.pallas import tpu as pltpu
from jax.experimental.pallas import tpu_sc as plsc


@functools.lru_cache(maxsize=None)
def _make_emb_lookup(B, V, D, chunk):
    info = plsc.get_sparse_core_info()
    nc, ns = info.num_cores, info.num_subcores
    nw = nc * ns
    b_per_w = B // nw
    n_chunks = b_per_w // chunk
    mesh = plsc.VectorSubcoreMesh(core_axis_name="c", subcore_axis_name="s")

    @functools.partial(
        pl.kernel,
        mesh=mesh,
        compiler_params=pltpu.CompilerParams(use_tc_tiling_on_sc=False),
        out_type=jax.ShapeDtypeStruct((B, D), jnp.float32),
        scratch_types=[
            pltpu.VMEM((chunk,), jnp.int32),
            pltpu.VMEM((chunk, D), jnp.float32),
            pltpu.SemaphoreType.DMA,
        ],
    )
    def emb(idx_hbm, table_hbm, out_hbm, idx_v, rows_v, sem):
        wid = lax.axis_index("s") * nc + lax.axis_index("c")
        base = wid * b_per_w

        def body(i, carry):
            off = base + i * chunk
            pltpu.sync_copy(idx_hbm.at[pl.ds(off, chunk)], idx_v)
            pltpu.async_copy(table_hbm.at[idx_v], rows_v, sem).wait()
            pltpu.sync_copy(rows_v, out_hbm.at[pl.ds(off, chunk)])
            return carry

        lax.fori_loop(0, n_chunks, body, 0)

    return emb


def kernel(x, table):
    bsz, seq = x.shape
    V, D = table.shape
    B = bsz * seq
    xf = x.reshape(B)
    out = _make_emb_lookup(B, V, D, 1024)(xf, table)
    return out.reshape(bsz, seq, D)


# speedup vs baseline: 1.0164x; 1.0164x over previous
import functools

import jax
import jax.numpy as jnp
from jax import lax
from jax.experimental import pallas as pl
from jax.experimental.pallas import tpu as pltpu
from jax.experimental.pallas import tpu_sc as plsc


@functools.lru_cache(maxsize=None)
def _make_emb_lookup(B, V, D, chunk):
    info = plsc.get_sparse_core_info()
    nc, ns = info.num_cores, info.num_subcores
    nw = nc * ns
    b_per_w = B // nw
    n_chunks = b_per_w // chunk
    assert n_chunks * chunk == b_per_w and n_chunks % 2 == 0 and n_chunks >= 4
    mesh = plsc.VectorSubcoreMesh(core_axis_name="c", subcore_axis_name="s")

    @functools.partial(
        pl.kernel,
        mesh=mesh,
        compiler_params=pltpu.CompilerParams(use_tc_tiling_on_sc=False),
        out_type=jax.ShapeDtypeStruct((B, D), jnp.float32),
        scratch_types=[
            pltpu.VMEM((n_chunks, chunk), jnp.int32),
            pltpu.VMEM((2, chunk, D), jnp.float32),
            pltpu.SemaphoreType.DMA,
            pltpu.SemaphoreType.DMA,
            pltpu.SemaphoreType.DMA,
            pltpu.SemaphoreType.DMA,
        ],
    )
    def emb(idx_hbm, table_hbm, out_hbm, idx_v, rows_v, sg0, sg1, so0, so1):
        wid = lax.axis_index("s") * nc + lax.axis_index("c")
        base = wid * b_per_w
        sg = (sg0, sg1)
        so = (so0, so1)

        def gather(i, b):
            return pltpu.make_async_copy(table_hbm.at[idx_v.at[i]], rows_v.at[b], sg[b])

        def out_copy(i, b):
            return pltpu.make_async_copy(
                rows_v.at[b], out_hbm.at[pl.ds(base + i * chunk, chunk)], so[b]
            )

        pltpu.sync_copy(idx_hbm.at[wid], idx_v)

        gather(0, 0).start()
        gather(1, 1).start()
        gather(0, 0).wait()
        out_copy(0, 0).start()

        def body(j, carry):
            i0 = 2 + 2 * j
            for b in range(2):
                i = i0 + b
                out_copy(i - 2, b).wait()
                gather(i, b).start()
                gather(i - 1, 1 - b).wait()
                out_copy(i - 1, 1 - b).start()
            return carry

        lax.fori_loop(0, (n_chunks - 2) // 2, body, 0)

        gather(n_chunks - 1, 1).wait()
        out_copy(n_chunks - 1, 1).start()
        out_copy(n_chunks - 2, 0).wait()
        out_copy(n_chunks - 1, 1).wait()

    return emb


def kernel(x, table):
    bsz, seq = x.shape
    V, D = table.shape
    B = bsz * seq
    chunk = 800
    info = plsc.get_sparse_core_info()
    nw = info.num_cores * info.num_subcores
    xf = x.reshape(nw, (B // nw) // chunk, chunk)
    out = _make_emb_lookup(B, V, D, chunk)(xf, table)
    return out.reshape(bsz, seq, D)
